# baseline (speedup 1.0000x reference)
import functools

import jax
import jax.numpy as jnp
from jax import lax
from jax.experimental import pallas as pl
from jax.experimental.pallas import tpu as pltpu
from jax.experimental.pallas import tpu_sc as plsc

DIM = 256
N_EMBED = 8192
N_TOKENS = 16 * 1024


NB = 256
MC = 512
N_GRID = N_TOKENS // NB
N_CHUNK = N_EMBED // MC


def _argmin_body(x_ref, e_ref, ind_ref):
    x = x_ref[...]
    x2 = jnp.sum(x * x, axis=1, keepdims=True)

    def step(m, carry):
        best, besti = carry
        off = pl.multiple_of(m * MC, MC)
        e = e_ref[:, pl.ds(off, MC)]
        e2 = jnp.sum(e * e, axis=0, keepdims=True)
        mm = jnp.dot(x, e, preferred_element_type=jnp.float32)
        dist = x2 - 2.0 * mm + e2
        cmin = jnp.min(dist, axis=1, keepdims=True)
        is_min = dist == cmin
        col = lax.broadcasted_iota(jnp.int32, (NB, MC), 1)
        cidx = jnp.min(jnp.where(is_min, col, N_EMBED), axis=1,
                       keepdims=True) + m * MC
        upd = cmin < best
        return jnp.where(upd, cmin, best), jnp.where(upd, cidx, besti)

    init = (jnp.full((NB, 1), jnp.inf, jnp.float32),
            jnp.zeros((NB, 1), jnp.int32))
    _, besti = lax.fori_loop(0, N_CHUNK, step, init)
    ind_ref[...] = besti


def _argmin_call(flat, embed):
    return pl.pallas_call(
        _argmin_body,
        grid=(N_GRID,),
        in_specs=[
            pl.BlockSpec((NB, DIM), lambda i: (i, 0)),
            pl.BlockSpec((DIM, N_EMBED), lambda i: (0, 0)),
        ],
        out_specs=pl.BlockSpec((NB, 1), lambda i: (i, 0)),
        out_shape=jax.ShapeDtypeStruct((N_TOKENS, 1), jnp.int32),
        compiler_params=pltpu.CompilerParams(
            dimension_semantics=("arbitrary",)),
    )(flat, embed)


SC_NC = 2
SC_NS = 16
NW = SC_NC * SC_NS
B_PER_W = N_TOKENS // NW
QC = 128
NCH = B_PER_W // QC


def _gather_call(table, idx):
    mesh = plsc.VectorSubcoreMesh(core_axis_name="c", subcore_axis_name="s")

    @functools.partial(
        pl.kernel, mesh=mesh,
        out_type=jax.ShapeDtypeStruct((N_TOKENS, DIM), jnp.float32),
        scratch_types=[
            pltpu.VMEM((B_PER_W,), jnp.int32),
            pltpu.VMEM((QC, DIM), jnp.float32),
            pltpu.VMEM((QC, DIM), jnp.float32),
            pltpu.SemaphoreType.DMA,
            pltpu.SemaphoreType.DMA,
        ],
    )
    def k(table_hbm, idx_hbm, out_hbm, idx_v, buf0, buf1, sem0, sem1):
        wid = lax.axis_index("s") * SC_NC + lax.axis_index("c")
        base = wid * B_PER_W
        pltpu.sync_copy(idx_hbm.at[pl.ds(base, B_PER_W)], idx_v)
        bufs, sems = (buf0, buf1), (sem0, sem1)
        cps = [pltpu.async_copy(table_hbm.at[idx_v.at[pl.ds(0, QC)]],
                                buf0, sem0), None]
        for c in range(NCH):
            if c + 1 < NCH:
                cps[(c + 1) % 2] = pltpu.async_copy(
                    table_hbm.at[idx_v.at[pl.ds((c + 1) * QC, QC)]],
                    bufs[(c + 1) % 2], sems[(c + 1) % 2])
            cps[c % 2].wait()
            pltpu.sync_copy(bufs[c % 2], out_hbm.at[pl.ds(base + c * QC, QC)])

    return k(table, idx)


K3_NB = 1024
K3_GRID = N_TOKENS // K3_NB


def _st_body(x_ref, q_ref, st_ref, acc_ref):
    i = pl.program_id(0)
    x = x_ref[...]
    q = q_ref[...]
    d = q - x
    st_ref[...] = x + d

    @pl.when(i == 0)
    def _():
        acc_ref[0, 0] = 0.0

    acc_ref[0, 0] += jnp.sum(d * d)

    @pl.when(i == K3_GRID - 1)
    def _():
        acc_ref[0, 0] = acc_ref[0, 0] * (1.0 / (N_TOKENS * DIM))


def _st_call(flat, q):
    return pl.pallas_call(
        _st_body,
        grid=(K3_GRID,),
        in_specs=[
            pl.BlockSpec((K3_NB, DIM), lambda i: (i, 0)),
            pl.BlockSpec((K3_NB, DIM), lambda i: (i, 0)),
        ],
        out_specs=[
            pl.BlockSpec((K3_NB, DIM), lambda i: (i, 0)),
            pl.BlockSpec(memory_space=pltpu.SMEM),
        ],
        out_shape=[
            jax.ShapeDtypeStruct((N_TOKENS, DIM), jnp.float32),
            jax.ShapeDtypeStruct((1, 1), jnp.float32),
        ],
        compiler_params=pltpu.CompilerParams(
            dimension_semantics=("arbitrary",)),
    )(flat, q)


def kernel(input, embed):
    flat = input.reshape(-1, DIM)
    ind2d = _argmin_call(flat, embed)
    table = jnp.asarray(embed.T)
    q = _gather_call(table, ind2d[:, 0])
    st, diffsum = _st_call(flat, q)
    quantize = st.reshape(input.shape)
    diff = diffsum.reshape(())
    embed_ind = ind2d[:, 0].reshape(input.shape[:-1])
    return (quantize, diff, embed_ind)


# speedup vs baseline: 1.0275x; 1.0275x over previous
import functools

import jax
import jax.numpy as jnp
from jax import lax
from jax.experimental import pallas as pl
from jax.experimental.pallas import tpu as pltpu
from jax.experimental.pallas import tpu_sc as plsc

DIM = 256
N_EMBED = 8192
N_TOKENS = 16 * 1024


NB = 256
MC = 512
N_GRID = N_TOKENS // NB
N_CHUNK = N_EMBED // MC


def _argmin_body(x_ref, e_ref, ind_ref):
    x = x_ref[...]

    def step(m, carry):
        best, besti = carry
        off = pl.multiple_of(m * MC, MC)
        e = e_ref[:, pl.ds(off, MC)]
        e2h = 0.5 * jnp.sum(e * e, axis=0, keepdims=True)
        mm = jnp.dot(x, e, preferred_element_type=jnp.float32)
        score = mm - e2h
        cmax = jnp.max(score, axis=1, keepdims=True)
        col = lax.broadcasted_iota(jnp.int32, (NB, MC), 1)
        cidx = jnp.min(jnp.where(score == cmax, col, N_EMBED), axis=1,
                       keepdims=True) + m * MC
        upd = cmax > best
        return jnp.where(upd, cmax, best), jnp.where(upd, cidx, besti)

    init = (jnp.full((NB, 1), -jnp.inf, jnp.float32),
            jnp.zeros((NB, 1), jnp.int32))
    _, besti = lax.fori_loop(0, N_CHUNK, step, init)
    ind_ref[...] = besti


def _argmin_call(flat, embed):
    return pl.pallas_call(
        _argmin_body,
        grid=(N_GRID,),
        in_specs=[
            pl.BlockSpec((NB, DIM), lambda i: (i, 0)),
            pl.BlockSpec((DIM, N_EMBED), lambda i: (0, 0)),
        ],
        out_specs=pl.BlockSpec((NB, 1), lambda i: (i, 0)),
        out_shape=jax.ShapeDtypeStruct((N_TOKENS, 1), jnp.int32),
        compiler_params=pltpu.CompilerParams(
            dimension_semantics=("arbitrary",)),
    )(flat, embed)


SC_NC = 2
SC_NS = 16
NW = SC_NC * SC_NS
B_PER_W = N_TOKENS // NW
QC = 128
NCH = B_PER_W // QC


def _gather_call(table, idx):
    mesh = plsc.VectorSubcoreMesh(core_axis_name="c", subcore_axis_name="s")

    @functools.partial(
        pl.kernel, mesh=mesh,
        out_type=jax.ShapeDtypeStruct((N_TOKENS, DIM), jnp.float32),
        scratch_types=[
            pltpu.VMEM((B_PER_W,), jnp.int32),
            pltpu.VMEM((QC, DIM), jnp.float32),
            pltpu.VMEM((QC, DIM), jnp.float32),
            pltpu.SemaphoreType.DMA,
            pltpu.SemaphoreType.DMA,
        ],
    )
    def k(table_hbm, idx_hbm, out_hbm, idx_v, buf0, buf1, sem0, sem1):
        wid = lax.axis_index("s") * SC_NC + lax.axis_index("c")
        base = wid * B_PER_W
        pltpu.sync_copy(idx_hbm.at[pl.ds(base, B_PER_W)], idx_v)
        bufs, sems = (buf0, buf1), (sem0, sem1)
        cps = [pltpu.async_copy(table_hbm.at[idx_v.at[pl.ds(0, QC)]],
                                buf0, sem0), None]
        for c in range(NCH):
            if c + 1 < NCH:
                cps[(c + 1) % 2] = pltpu.async_copy(
                    table_hbm.at[idx_v.at[pl.ds((c + 1) * QC, QC)]],
                    bufs[(c + 1) % 2], sems[(c + 1) % 2])
            cps[c % 2].wait()
            pltpu.sync_copy(bufs[c % 2], out_hbm.at[pl.ds(base + c * QC, QC)])

    return k(table, idx)


K3_NB = 1024
K3_GRID = N_TOKENS // K3_NB


def _st_body(x_ref, q_ref, st_ref, acc_ref):
    i = pl.program_id(0)
    x = x_ref[...]
    q = q_ref[...]
    d = q - x
    st_ref[...] = x + d

    @pl.when(i == 0)
    def _():
        acc_ref[0, 0] = 0.0

    acc_ref[0, 0] += jnp.sum(d * d)

    @pl.when(i == K3_GRID - 1)
    def _():
        acc_ref[0, 0] = acc_ref[0, 0] * (1.0 / (N_TOKENS * DIM))


def _st_call(flat, q):
    return pl.pallas_call(
        _st_body,
        grid=(K3_GRID,),
        in_specs=[
            pl.BlockSpec((K3_NB, DIM), lambda i: (i, 0)),
            pl.BlockSpec((K3_NB, DIM), lambda i: (i, 0)),
        ],
        out_specs=[
            pl.BlockSpec((K3_NB, DIM), lambda i: (i, 0)),
            pl.BlockSpec(memory_space=pltpu.SMEM),
        ],
        out_shape=[
            jax.ShapeDtypeStruct((N_TOKENS, DIM), jnp.float32),
            jax.ShapeDtypeStruct((1, 1), jnp.float32),
        ],
        compiler_params=pltpu.CompilerParams(
            dimension_semantics=("arbitrary",)),
    )(flat, q)


def kernel(input, embed):
    flat = input.reshape(-1, DIM)
    ind2d = _argmin_call(flat, embed)
    table = jnp.asarray(embed.T)
    q = _gather_call(table, ind2d[:, 0])
    st, diffsum = _st_call(flat, q)
    quantize = st.reshape(input.shape)
    diff = diffsum.reshape(())
    embed_ind = ind2d[:, 0].reshape(input.shape[:-1])
    return (quantize, diff, embed_ind)
